# Initial kernel scaffold; baseline (speedup 1.0000x reference)
import jax
import jax.numpy as jnp
from jax.experimental import pallas as pl


def kernel(h, adjs, W, attn_src, attn_trg):
    raise NotImplementedError("write your pallas kernel here")


# speedup vs baseline: 44.1385x; 44.1385x over previous
import functools

import jax
import jax.numpy as jnp
from jax import lax
from jax.experimental import pallas as pl
from jax.experimental.pallas import tpu as pltpu
from jax.experimental.pallas import tpu_sc as plsc

_B, _N, _E = 2, 50000, 800000
_IN_F, _HEADS, _OUT_F = 128, 4, 16

_BN = 2000
_NBLK = _N // _BN


def _proj_body(h_ref, w_ref, as_ref, at_ref, hp_ref, asrc_ref, atrg_ref, m_ref):
    hb = h_ref[0]
    s_cols, t_cols = [], []
    for hd in range(_HEADS):
        wh = w_ref[hd * _OUT_F:(hd + 1) * _OUT_F, :]
        hp_h = lax.dot_general(hb, wh, (((1,), (1,)), ((), ())),
                               preferred_element_type=jnp.float32)
        hp_ref[0, hd] = hp_h
        s_cols.append(jnp.sum(hp_h * as_ref[0, hd][None, :], axis=-1, keepdims=True))
        t_cols.append(jnp.sum(hp_h * at_ref[0, hd][None, :], axis=-1, keepdims=True))
    pad = jnp.zeros((_BN, 16 - _HEADS), jnp.float32)
    asv = jnp.concatenate(s_cols + [pad], axis=1)
    atv = jnp.concatenate(t_cols + [pad], axis=1)
    asrc_ref[0] = asv
    atrg_ref[0] = atv
    cur = jnp.stack([jnp.full((8, 128), jnp.max(jnp.concatenate(s_cols, axis=1)),
                              jnp.float32),
                     jnp.full((8, 128), jnp.max(jnp.concatenate(t_cols, axis=1)),
                              jnp.float32)])

    @pl.when(pl.program_id(1) == 0)
    def _():
        m_ref[0] = cur

    @pl.when(pl.program_id(1) != 0)
    def _():
        m_ref[0] = jnp.maximum(m_ref[0], cur)


_proj = pl.pallas_call(
    _proj_body,
    grid=(_B, _NBLK),
    in_specs=[
        pl.BlockSpec((1, _BN, _IN_F), lambda b, n: (b, n, 0)),
        pl.BlockSpec((_HEADS * _OUT_F, _IN_F), lambda b, n: (0, 0)),
        pl.BlockSpec((1, _HEADS, _OUT_F), lambda b, n: (0, 0, 0)),
        pl.BlockSpec((1, _HEADS, _OUT_F), lambda b, n: (0, 0, 0)),
    ],
    out_specs=[
        pl.BlockSpec((1, _HEADS, _BN, _OUT_F), lambda b, n: (b, 0, n, 0)),
        pl.BlockSpec((1, _BN, 16), lambda b, n: (b, n, 0)),
        pl.BlockSpec((1, _BN, 16), lambda b, n: (b, n, 0)),
        pl.BlockSpec((1, 2, 8, 128), lambda b, n: (b, 0, 0, 0)),
    ],
    out_shape=[
        jax.ShapeDtypeStruct((_B, _HEADS, _N, _OUT_F), jnp.float32),
        jax.ShapeDtypeStruct((_B, _N, 16), jnp.float32),
        jax.ShapeDtypeStruct((_B, _N, 16), jnp.float32),
        jax.ShapeDtypeStruct((_B, 2, 8, 128), jnp.float32),
    ],
)

_NPAD = 50048
_ECH = _E // 128
_NCH = _NPAD // 128
_NFULL = _N // 128
_TAIL = _N - _NFULL * 128
_TAIL_TILE = _NFULL % 16

_mesh = plsc.VectorSubcoreMesh(core_axis_name="c", subcore_axis_name="s")


@functools.partial(
    pl.kernel,
    out_type=(
        jax.ShapeDtypeStruct((_B * _HEADS * _N, _OUT_F), jnp.float32),
        jax.ShapeDtypeStruct((_B * _HEADS * _E,), jnp.float32),
    ),
    mesh=_mesh,
    compiler_params=pltpu.CompilerParams(needs_layout_passes=False,
                                         use_tc_tiling_on_sc=False),
    scratch_types=[
        pltpu.VMEM_SHARED((_NPAD, 16), jnp.float32),
        pltpu.VMEM_SHARED((_NPAD, _OUT_F), jnp.float32),
        pltpu.VMEM((128,), jnp.int32),
        pltpu.VMEM((128,), jnp.int32),
        pltpu.VMEM((128,), jnp.int32),
        pltpu.VMEM((128, 16), jnp.float32),
        pltpu.VMEM((128, 16), jnp.float32),
        pltpu.VMEM((128, 16), jnp.float32),
        pltpu.VMEM((_HEADS, 128), jnp.float32),
        pltpu.VMEM((128,), jnp.float32),
        pltpu.VMEM((128, _OUT_F), jnp.float32),
        pltpu.VMEM((128, _OUT_F), jnp.float32),
        pltpu.VMEM((128, 16), jnp.float32),
        pltpu.VMEM((32,), jnp.float32),
        pltpu.VMEM((128, 16), jnp.float32),
    ],
)
def _sc_gat(hp_hbm, asrc_hbm, atrg_hbm, src_hbm, trg_hbm, m_hbm,
            out_hbm, exp_hbm, denom_sp, agg_sp, srcb, trgb, offb,
            sbuf, tbuf, ebuf, ehead, ewt, hpbuf, abuf, dbuf, mbuf, z16):
    b = lax.axis_index("c")
    s = lax.axis_index("s")
    iota = lax.iota(jnp.int32, 16)
    r0 = iota >> 2
    c0 = iota & 3
    zv = jnp.zeros((16,), jnp.float32)

    def _zfill(i, carry):
        z16[i, :] = zv
        ebuf[i, :] = zv
        return carry

    lax.fori_loop(0, 128, _zfill, 0)

    pltpu.sync_copy(m_hbm.at[pl.ds(b * 32, 32)], mbuf)
    msum = mbuf[pl.ds(0, 16)] + mbuf[pl.ds(16, 16)]
    mvec = jnp.maximum(msum, 0.2 * msum)

    def _zden(j, carry):
        g = j * 16 + s

        @pl.when(g < _NCH)
        def _():
            pltpu.sync_copy(z16, denom_sp.at[pl.ds(g * 128, 128)])
        return carry

    lax.fori_loop(0, 25, _zden, 0)
    plsc.subcore_barrier()

    def _p1(j, carry):
        g = j * 16 + s

        @pl.when(g < _ECH)
        def _():
            base = b * _E + g * 128
            pltpu.sync_copy(src_hbm.at[pl.ds(base, 128)], srcb)
            pltpu.sync_copy(trg_hbm.at[pl.ds(base, 128)], trgb)
            noff = b * _N
            for k in range(8):
                offb[pl.ds(k * 16, 16)] = srcb[pl.ds(k * 16, 16)] + noff
            pltpu.sync_copy(asrc_hbm.at[offb], sbuf)
            for k in range(8):
                offb[pl.ds(k * 16, 16)] = trgb[pl.ds(k * 16, 16)] + noff
            pltpu.sync_copy(atrg_hbm.at[offb], tbuf)
            for k in range(32):
                rows = r0 + 4 * k
                sv = plsc.load_gather(sbuf, [rows, c0])
                tv = plsc.load_gather(tbuf, [rows, c0])
                x = sv + tv
                ev = jnp.exp(jnp.maximum(x, 0.2 * x) - mvec)
                plsc.store_scatter(ebuf, [rows, c0], ev)
            pltpu.sync_copy(ebuf, denom_sp.at[trgb], add=True)
            for hd in range(_HEADS):
                hsel = jnp.full((16,), hd, jnp.int32)
                for k in range(8):
                    ehead[hd, pl.ds(k * 16, 16)] = plsc.load_gather(
                        ebuf, [iota + 16 * k, hsel])
                pltpu.sync_copy(
                    ehead.at[hd],
                    exp_hbm.at[pl.ds((b * _HEADS + hd) * _E + g * 128, 128)])
        return carry

    lax.fori_loop(0, 391, _p1, 0)
    plsc.subcore_barrier()

    def _head_pass(hd, hcarry):
        def _zagg(j, carry):
            g = j * 16 + s

            @pl.when(g < _NCH)
            def _():
                pltpu.sync_copy(z16, agg_sp.at[pl.ds(g * 128, 128)])
            return carry

        lax.fori_loop(0, 25, _zagg, 0)
        plsc.subcore_barrier()

        def _p2(j, carry):
            g = j * 16 + s

            @pl.when(g < _ECH)
            def _():
                base = b * _E + g * 128
                pltpu.sync_copy(src_hbm.at[pl.ds(base, 128)], srcb)
                pltpu.sync_copy(trg_hbm.at[pl.ds(base, 128)], trgb)
                pltpu.sync_copy(
                    exp_hbm.at[pl.ds((b * _HEADS + hd) * _E + g * 128, 128)], ewt)
                hoff = (b * _HEADS + hd) * _N
                for k in range(8):
                    offb[pl.ds(k * 16, 16)] = srcb[pl.ds(k * 16, 16)] + hoff
                pltpu.sync_copy(hp_hbm.at[offb], hpbuf)
                def _wmul(e_i, carry):
                    wv = plsc.load_gather(ewt, [jnp.full((16,), e_i, jnp.int32)])
                    hpbuf[e_i, :] = hpbuf[e_i, :] * wv
                    return carry

                lax.fori_loop(0, 128, _wmul, 0, unroll=8)
                pltpu.sync_copy(hpbuf, agg_sp.at[trgb], add=True)
            return carry

        lax.fori_loop(0, 391, _p2, 0)
        plsc.subcore_barrier()

        def _fin(j, carry):
            g = j * 16 + s

            @pl.when(g < _NFULL)
            def _():
                nb = g * 128
                pltpu.sync_copy(agg_sp.at[pl.ds(nb, 128)], abuf)
                pltpu.sync_copy(denom_sp.at[pl.ds(nb, 128)], dbuf)
                hsel = jnp.full((16,), hd, jnp.int32)

                def _ndiv(i, carry):
                    dv = plsc.load_gather(dbuf, [jnp.full((16,), i, jnp.int32), hsel])
                    abuf[i, :] = abuf[i, :] / (dv + 1e-16)
                    return carry

                lax.fori_loop(0, 128, _ndiv, 0, unroll=8)
                pltpu.sync_copy(
                    abuf, out_hbm.at[pl.ds((b * _HEADS + hd) * _N + nb, 128)])
            return carry

        lax.fori_loop(0, 25, _fin, 0)

        @pl.when(s == _TAIL_TILE)
        def _():
            nb = _NFULL * 128
            pltpu.sync_copy(agg_sp.at[pl.ds(nb, _TAIL)], abuf.at[pl.ds(0, _TAIL)])
            pltpu.sync_copy(denom_sp.at[pl.ds(nb, _TAIL)], dbuf.at[pl.ds(0, _TAIL)])
            hsel = jnp.full((16,), hd, jnp.int32)

            def _ndiv2(i, carry):
                dv = plsc.load_gather(dbuf, [jnp.full((16,), i, jnp.int32), hsel])
                abuf[i, :] = abuf[i, :] / (dv + 1e-16)
                return carry

            lax.fori_loop(0, _TAIL, _ndiv2, 0, unroll=8)
            pltpu.sync_copy(
                abuf.at[pl.ds(0, _TAIL)],
                out_hbm.at[pl.ds((b * _HEADS + hd) * _N + nb, _TAIL)])

        plsc.subcore_barrier()
        return hcarry

    lax.fori_loop(0, _HEADS, _head_pass, 0)


def kernel(h, adjs, W, attn_src, attn_trg):
    src = adjs[:, 0, :].astype(jnp.int32).reshape(_B * _E)
    trg = adjs[:, 1, :].astype(jnp.int32).reshape(_B * _E)
    hp4, asrc, atrg, mpair = _proj(h, W, attn_src, attn_trg)
    out_flat, _ = _sc_gat(
        hp4.reshape(_B * _HEADS * _N, _OUT_F),
        asrc.reshape(_B * _N, 16),
        atrg.reshape(_B * _N, 16),
        src, trg,
        mpair[:, :, 0, :16].reshape(_B * 32),
    )
    return out_flat.reshape(_B, _HEADS, _N, _OUT_F)


# speedup vs baseline: 60.8110x; 1.3777x over previous
import functools

import jax
import jax.numpy as jnp
from jax import lax
from jax.experimental import pallas as pl
from jax.experimental.pallas import tpu as pltpu
from jax.experimental.pallas import tpu_sc as plsc

_B, _N, _E = 2, 50000, 800000
_IN_F, _HEADS, _OUT_F = 128, 4, 16

_BN = 2000
_NBLK = _N // _BN


def _proj_body(h_ref, w_ref, as_ref, at_ref, hp_ref, asrc_ref, atrg_ref, m_ref):
    hb = h_ref[0]
    s_cols, t_cols = [], []
    for hd in range(_HEADS):
        wh = w_ref[hd * _OUT_F:(hd + 1) * _OUT_F, :]
        hp_h = lax.dot_general(hb, wh, (((1,), (1,)), ((), ())),
                               preferred_element_type=jnp.float32)
        hp_ref[0, hd] = hp_h
        s_cols.append(jnp.sum(hp_h * as_ref[0, hd][None, :], axis=-1, keepdims=True))
        t_cols.append(jnp.sum(hp_h * at_ref[0, hd][None, :], axis=-1, keepdims=True))
    pad = jnp.zeros((_BN, 16 - _HEADS), jnp.float32)
    asv = jnp.concatenate(s_cols + [pad], axis=1)
    atv = jnp.concatenate(t_cols + [pad], axis=1)
    asrc_ref[0] = asv
    atrg_ref[0] = atv
    cur = jnp.stack([jnp.full((8, 128), jnp.max(jnp.concatenate(s_cols, axis=1)),
                              jnp.float32),
                     jnp.full((8, 128), jnp.max(jnp.concatenate(t_cols, axis=1)),
                              jnp.float32)])

    @pl.when(pl.program_id(1) == 0)
    def _():
        m_ref[0] = cur

    @pl.when(pl.program_id(1) != 0)
    def _():
        m_ref[0] = jnp.maximum(m_ref[0], cur)


_proj = pl.pallas_call(
    _proj_body,
    grid=(_B, _NBLK),
    in_specs=[
        pl.BlockSpec((1, _BN, _IN_F), lambda b, n: (b, n, 0)),
        pl.BlockSpec((_HEADS * _OUT_F, _IN_F), lambda b, n: (0, 0)),
        pl.BlockSpec((1, _HEADS, _OUT_F), lambda b, n: (0, 0, 0)),
        pl.BlockSpec((1, _HEADS, _OUT_F), lambda b, n: (0, 0, 0)),
    ],
    out_specs=[
        pl.BlockSpec((1, _HEADS, _BN, _OUT_F), lambda b, n: (b, 0, n, 0)),
        pl.BlockSpec((1, _BN, 16), lambda b, n: (b, n, 0)),
        pl.BlockSpec((1, _BN, 16), lambda b, n: (b, n, 0)),
        pl.BlockSpec((1, 2, 8, 128), lambda b, n: (b, 0, 0, 0)),
    ],
    out_shape=[
        jax.ShapeDtypeStruct((_B, _HEADS, _N, _OUT_F), jnp.float32),
        jax.ShapeDtypeStruct((_B, _N, 16), jnp.float32),
        jax.ShapeDtypeStruct((_B, _N, 16), jnp.float32),
        jax.ShapeDtypeStruct((_B, 2, 8, 128), jnp.float32),
    ],
)

_NPAD = 50048
_ECH = _E // 128
_NCH = _NPAD // 128
_NFULL = _N // 128
_TAIL = _N - _NFULL * 128
_TAIL_TILE = _NFULL % 16

_mesh = plsc.VectorSubcoreMesh(core_axis_name="c", subcore_axis_name="s")


@functools.partial(
    pl.kernel,
    out_type=(
        jax.ShapeDtypeStruct((_B * _HEADS * _N, _OUT_F), jnp.float32),
        jax.ShapeDtypeStruct((_B * _HEADS * _E,), jnp.float32),
    ),
    mesh=_mesh,
    compiler_params=pltpu.CompilerParams(needs_layout_passes=False,
                                         use_tc_tiling_on_sc=False),
    scratch_types=[
        pltpu.VMEM_SHARED((_NPAD, 16), jnp.float32),
        pltpu.VMEM_SHARED((_NPAD, _OUT_F), jnp.float32),
        pltpu.VMEM((128,), jnp.int32),
        pltpu.VMEM((128,), jnp.int32),
        pltpu.VMEM((128,), jnp.int32),
        pltpu.VMEM((128, 16), jnp.float32),
        pltpu.VMEM((128, 16), jnp.float32),
        pltpu.VMEM((128, 16), jnp.float32),
        pltpu.VMEM((_HEADS, 128), jnp.float32),
        pltpu.VMEM((128,), jnp.float32),
        pltpu.VMEM((128, _OUT_F), jnp.float32),
        pltpu.VMEM((128, _OUT_F), jnp.float32),
        pltpu.VMEM((128, 16), jnp.float32),
        pltpu.VMEM((32,), jnp.float32),
        pltpu.VMEM((128, 16), jnp.float32),
        pltpu.VMEM((128,), jnp.int32),
        pltpu.SemaphoreType.DMA,
        pltpu.SemaphoreType.DMA,
        pltpu.SemaphoreType.DMA,
    ],
)
def _sc_gat(hp_hbm, asrc_hbm, atrg_hbm, src_hbm, trg_hbm, m_hbm,
            out_hbm, exp_hbm, denom_sp, agg_sp, srcb, trgb, offb,
            sbuf, tbuf, ebuf, ehead, ewt, hpbuf, abuf, dbuf, mbuf, z16,
            offb2, semi, semg, semsc):
    b = lax.axis_index("c")
    s = lax.axis_index("s")
    iota = lax.iota(jnp.int32, 16)
    r0 = iota >> 2
    c0 = iota & 3
    zv = jnp.zeros((16,), jnp.float32)

    def _zfill(i, carry):
        z16[i, :] = zv
        ebuf[i, :] = zv
        return carry

    lax.fori_loop(0, 128, _zfill, 0)

    pltpu.sync_copy(m_hbm.at[pl.ds(b * 32, 32)], mbuf)
    msum = mbuf[pl.ds(0, 16)] + mbuf[pl.ds(16, 16)]
    mvec = jnp.maximum(msum, 0.2 * msum)

    def _zden(j, carry):
        g = j * 16 + s

        @pl.when(g < _NCH)
        def _():
            pltpu.sync_copy(z16, denom_sp.at[pl.ds(g * 128, 128)])
        return carry

    lax.fori_loop(0, 25, _zden, 0)
    plsc.subcore_barrier()

    def _p1(j, carry):
        g = j * 16 + s

        @pl.when(g < _ECH)
        def _():
            base = b * _E + g * 128
            d_s = pltpu.async_copy(src_hbm.at[pl.ds(base, 128)], srcb, semi)
            d_t = pltpu.async_copy(trg_hbm.at[pl.ds(base, 128)], trgb, semi)
            d_s.wait()
            d_t.wait()
            noff = b * _N
            for k in range(8):
                offb[pl.ds(k * 16, 16)] = srcb[pl.ds(k * 16, 16)] + noff
                offb2[pl.ds(k * 16, 16)] = trgb[pl.ds(k * 16, 16)] + noff
            d_a = pltpu.async_copy(asrc_hbm.at[offb], sbuf, semg)
            d_b = pltpu.async_copy(atrg_hbm.at[offb2], tbuf, semg)
            d_a.wait()
            d_b.wait()
            for k in range(32):
                rows = r0 + 4 * k
                sv = plsc.load_gather(sbuf, [rows, c0])
                tv = plsc.load_gather(tbuf, [rows, c0])
                x = sv + tv
                ev = jnp.exp(jnp.maximum(x, 0.2 * x) - mvec)
                plsc.store_scatter(ebuf, [rows, c0], ev)
            d_d = pltpu.async_copy(ebuf, denom_sp.at[trgb], semsc, add=True)
            d_es = []
            for hd in range(_HEADS):
                hsel = jnp.full((16,), hd, jnp.int32)
                for k in range(8):
                    ehead[hd, pl.ds(k * 16, 16)] = plsc.load_gather(
                        ebuf, [iota + 16 * k, hsel])
                d_es.append(pltpu.async_copy(
                    ehead.at[hd],
                    exp_hbm.at[pl.ds((b * _HEADS + hd) * _E + g * 128, 128)],
                    semg))
            d_d.wait()
            for d_e in d_es:
                d_e.wait()
        return carry

    lax.fori_loop(0, 391, _p1, 0)
    plsc.subcore_barrier()

    def _head_pass(hd, hcarry):
        def _zagg(j, carry):
            g = j * 16 + s

            @pl.when(g < _NCH)
            def _():
                pltpu.sync_copy(z16, agg_sp.at[pl.ds(g * 128, 128)])
            return carry

        lax.fori_loop(0, 25, _zagg, 0)
        plsc.subcore_barrier()

        def _p2(j, carry):
            g = j * 16 + s

            @pl.when(g < _ECH)
            def _():
                base = b * _E + g * 128
                d_s = pltpu.async_copy(src_hbm.at[pl.ds(base, 128)], srcb, semi)
                d_t = pltpu.async_copy(trg_hbm.at[pl.ds(base, 128)], trgb, semi)
                d_w = pltpu.async_copy(
                    exp_hbm.at[pl.ds((b * _HEADS + hd) * _E + g * 128, 128)],
                    ewt, semi)
                d_s.wait()
                d_t.wait()
                d_w.wait()
                hoff = (b * _HEADS + hd) * _N
                for k in range(8):
                    offb[pl.ds(k * 16, 16)] = srcb[pl.ds(k * 16, 16)] + hoff
                pltpu.sync_copy(hp_hbm.at[offb], hpbuf)
                def _wmul(e_i, carry):
                    wv = plsc.load_gather(ewt, [jnp.full((16,), e_i, jnp.int32)])
                    hpbuf[e_i, :] = hpbuf[e_i, :] * wv
                    return carry

                lax.fori_loop(0, 128, _wmul, 0, unroll=8)
                pltpu.sync_copy(hpbuf, agg_sp.at[trgb], add=True)
            return carry

        lax.fori_loop(0, 391, _p2, 0)
        plsc.subcore_barrier()

        def _fin(j, carry):
            g = j * 16 + s

            @pl.when(g < _NFULL)
            def _():
                nb = g * 128
                pltpu.sync_copy(agg_sp.at[pl.ds(nb, 128)], abuf)
                pltpu.sync_copy(denom_sp.at[pl.ds(nb, 128)], dbuf)
                hsel = jnp.full((16,), hd, jnp.int32)

                def _ndiv(i, carry):
                    dv = plsc.load_gather(dbuf, [jnp.full((16,), i, jnp.int32), hsel])
                    abuf[i, :] = abuf[i, :] / (dv + 1e-16)
                    return carry

                lax.fori_loop(0, 128, _ndiv, 0, unroll=8)
                pltpu.sync_copy(
                    abuf, out_hbm.at[pl.ds((b * _HEADS + hd) * _N + nb, 128)])
            return carry

        lax.fori_loop(0, 25, _fin, 0)

        @pl.when(s == _TAIL_TILE)
        def _():
            nb = _NFULL * 128
            pltpu.sync_copy(agg_sp.at[pl.ds(nb, _TAIL)], abuf.at[pl.ds(0, _TAIL)])
            pltpu.sync_copy(denom_sp.at[pl.ds(nb, _TAIL)], dbuf.at[pl.ds(0, _TAIL)])
            hsel = jnp.full((16,), hd, jnp.int32)

            def _ndiv2(i, carry):
                dv = plsc.load_gather(dbuf, [jnp.full((16,), i, jnp.int32), hsel])
                abuf[i, :] = abuf[i, :] / (dv + 1e-16)
                return carry

            lax.fori_loop(0, _TAIL, _ndiv2, 0, unroll=8)
            pltpu.sync_copy(
                abuf.at[pl.ds(0, _TAIL)],
                out_hbm.at[pl.ds((b * _HEADS + hd) * _N + nb, _TAIL)])

        plsc.subcore_barrier()
        return hcarry

    lax.fori_loop(0, _HEADS, _head_pass, 0)


def kernel(h, adjs, W, attn_src, attn_trg):
    src = adjs[:, 0, :].astype(jnp.int32).reshape(_B * _E)
    trg = adjs[:, 1, :].astype(jnp.int32).reshape(_B * _E)
    hp4, asrc, atrg, mpair = _proj(h, W, attn_src, attn_trg)
    out_flat, _ = _sc_gat(
        hp4.reshape(_B * _HEADS * _N, _OUT_F),
        asrc.reshape(_B * _N, 16),
        atrg.reshape(_B * _N, 16),
        src, trg,
        mpair[:, :, 0, :16].reshape(_B * 32),
    )
    return out_flat.reshape(_B, _HEADS, _N, _OUT_F)


# speedup vs baseline: 77.7516x; 1.2786x over previous
import functools

import jax
import jax.numpy as jnp
from jax import lax
from jax.experimental import pallas as pl
from jax.experimental.pallas import tpu as pltpu
from jax.experimental.pallas import tpu_sc as plsc

_B, _N, _E = 2, 50000, 800000
_IN_F, _HEADS, _OUT_F = 128, 4, 16

_BN = 2000
_NBLK = _N // _BN


def _proj_body(h_ref, w_ref, as_ref, at_ref, hp_ref, asrc_ref, atrg_ref, m_ref):
    hb = h_ref[0]
    s_cols, t_cols = [], []
    for hd in range(_HEADS):
        wh = w_ref[hd * _OUT_F:(hd + 1) * _OUT_F, :]
        hp_h = lax.dot_general(hb, wh, (((1,), (1,)), ((), ())),
                               preferred_element_type=jnp.float32)
        hp_ref[0, hd] = hp_h
        s_cols.append(jnp.sum(hp_h * as_ref[0, hd][None, :], axis=-1, keepdims=True))
        t_cols.append(jnp.sum(hp_h * at_ref[0, hd][None, :], axis=-1, keepdims=True))
    pad = jnp.zeros((_BN, 16 - _HEADS), jnp.float32)
    asv = jnp.concatenate(s_cols + [pad], axis=1)
    atv = jnp.concatenate(t_cols + [pad], axis=1)
    asrc_ref[0] = asv
    atrg_ref[0] = atv
    cur = jnp.stack([jnp.full((8, 128), jnp.max(jnp.concatenate(s_cols, axis=1)),
                              jnp.float32),
                     jnp.full((8, 128), jnp.max(jnp.concatenate(t_cols, axis=1)),
                              jnp.float32)])

    @pl.when(pl.program_id(1) == 0)
    def _():
        m_ref[0] = cur

    @pl.when(pl.program_id(1) != 0)
    def _():
        m_ref[0] = jnp.maximum(m_ref[0], cur)


_proj = pl.pallas_call(
    _proj_body,
    grid=(_B, _NBLK),
    in_specs=[
        pl.BlockSpec((1, _BN, _IN_F), lambda b, n: (b, n, 0)),
        pl.BlockSpec((_HEADS * _OUT_F, _IN_F), lambda b, n: (0, 0)),
        pl.BlockSpec((1, _HEADS, _OUT_F), lambda b, n: (0, 0, 0)),
        pl.BlockSpec((1, _HEADS, _OUT_F), lambda b, n: (0, 0, 0)),
    ],
    out_specs=[
        pl.BlockSpec((1, _HEADS, _BN, _OUT_F), lambda b, n: (b, 0, n, 0)),
        pl.BlockSpec((1, _BN, 16), lambda b, n: (b, n, 0)),
        pl.BlockSpec((1, _BN, 16), lambda b, n: (b, n, 0)),
        pl.BlockSpec((1, 2, 8, 128), lambda b, n: (b, 0, 0, 0)),
    ],
    out_shape=[
        jax.ShapeDtypeStruct((_B, _HEADS, _N, _OUT_F), jnp.float32),
        jax.ShapeDtypeStruct((_B, _N, 16), jnp.float32),
        jax.ShapeDtypeStruct((_B, _N, 16), jnp.float32),
        jax.ShapeDtypeStruct((_B, 2, 8, 128), jnp.float32),
    ],
)

_NPAD = 50048
_ECH = _E // 128
_NCH = _NPAD // 128
_NFULL = _N // 128
_TAIL = _N - _NFULL * 128
_TAIL_TILE = _NFULL % 16

_mesh = plsc.VectorSubcoreMesh(core_axis_name="c", subcore_axis_name="s")


@functools.partial(
    pl.kernel,
    out_type=(
        jax.ShapeDtypeStruct((_B * _HEADS * _N, _OUT_F), jnp.float32),
        jax.ShapeDtypeStruct((_B * _HEADS * _E,), jnp.float32),
    ),
    mesh=_mesh,
    compiler_params=pltpu.CompilerParams(needs_layout_passes=False,
                                         use_tc_tiling_on_sc=False),
    scratch_types=[
        pltpu.VMEM_SHARED((_NPAD, 16), jnp.float32),
        pltpu.VMEM_SHARED((_NPAD, _OUT_F), jnp.float32),
        pltpu.VMEM((128,), jnp.int32),
        pltpu.VMEM((128,), jnp.int32),
        pltpu.VMEM((128,), jnp.int32),
        pltpu.VMEM((128, 16), jnp.float32),
        pltpu.VMEM((128, 16), jnp.float32),
        pltpu.VMEM((128, 16), jnp.float32),
        pltpu.VMEM((_HEADS, 128), jnp.float32),
        pltpu.VMEM((128,), jnp.float32),
        pltpu.VMEM((128, _OUT_F), jnp.float32),
        pltpu.VMEM((128, _OUT_F), jnp.float32),
        pltpu.VMEM((128, 16), jnp.float32),
        pltpu.VMEM((32,), jnp.float32),
        pltpu.VMEM((128, 16), jnp.float32),
        pltpu.VMEM((128,), jnp.int32),
        pltpu.SemaphoreType.DMA,
        pltpu.SemaphoreType.DMA,
        pltpu.SemaphoreType.DMA,
        pltpu.VMEM((128,), jnp.int32),
        pltpu.VMEM((128,), jnp.int32),
        pltpu.VMEM((128,), jnp.float32),
        pltpu.VMEM((128, _OUT_F), jnp.float32),
        pltpu.SemaphoreType.DMA,
        pltpu.SemaphoreType.DMA,
        pltpu.SemaphoreType.DMA,
    ],
)
def _sc_gat(hp_hbm, asrc_hbm, atrg_hbm, src_hbm, trg_hbm, m_hbm,
            out_hbm, exp_hbm, denom_sp, agg_sp, srcb, trgb, offb,
            sbuf, tbuf, ebuf, ehead, ewt, hpbuf, abuf, dbuf, mbuf, z16,
            offb2, semi, semg, semsc,
            srcb2, trgb2, ewt2, hpbuf2, semi2, semg2, semsc2):
    b = lax.axis_index("c")
    s = lax.axis_index("s")
    iota = lax.iota(jnp.int32, 16)
    r0 = iota >> 2
    c0 = iota & 3
    zv = jnp.zeros((16,), jnp.float32)

    def _zfill(i, carry):
        z16[i, :] = zv
        ebuf[i, :] = zv
        return carry

    lax.fori_loop(0, 128, _zfill, 0)

    pltpu.sync_copy(m_hbm.at[pl.ds(b * 32, 32)], mbuf)
    msum = mbuf[pl.ds(0, 16)] + mbuf[pl.ds(16, 16)]
    mvec = jnp.maximum(msum, 0.2 * msum)

    def _zden(j, carry):
        g = j * 16 + s

        @pl.when(g < _NCH)
        def _():
            pltpu.sync_copy(z16, denom_sp.at[pl.ds(g * 128, 128)])
        return carry

    lax.fori_loop(0, 25, _zden, 0)
    plsc.subcore_barrier()

    def _p1(j, carry):
        g = j * 16 + s

        @pl.when(g < _ECH)
        def _():
            base = b * _E + g * 128
            d_s = pltpu.async_copy(src_hbm.at[pl.ds(base, 128)], srcb, semi)
            d_t = pltpu.async_copy(trg_hbm.at[pl.ds(base, 128)], trgb, semi)
            d_s.wait()
            d_t.wait()
            noff = b * _N
            for k in range(8):
                offb[pl.ds(k * 16, 16)] = srcb[pl.ds(k * 16, 16)] + noff
                offb2[pl.ds(k * 16, 16)] = trgb[pl.ds(k * 16, 16)] + noff
            d_a = pltpu.async_copy(asrc_hbm.at[offb], sbuf, semg)
            d_b = pltpu.async_copy(atrg_hbm.at[offb2], tbuf, semg)
            d_a.wait()
            d_b.wait()
            for k in range(32):
                rows = r0 + 4 * k
                sv = plsc.load_gather(sbuf, [rows, c0])
                tv = plsc.load_gather(tbuf, [rows, c0])
                x = sv + tv
                ev = jnp.exp(jnp.maximum(x, 0.2 * x) - mvec)
                plsc.store_scatter(ebuf, [rows, c0], ev)
            d_d = pltpu.async_copy(ebuf, denom_sp.at[trgb], semsc, add=True)
            d_es = []
            for hd in range(_HEADS):
                hsel = jnp.full((16,), hd, jnp.int32)
                for k in range(8):
                    ehead[hd, pl.ds(k * 16, 16)] = plsc.load_gather(
                        ebuf, [iota + 16 * k, hsel])
                d_es.append(pltpu.async_copy(
                    ehead.at[hd],
                    exp_hbm.at[pl.ds((b * _HEADS + hd) * _E + g * 128, 128)],
                    semg))
            d_d.wait()
            for d_e in d_es:
                d_e.wait()
        return carry

    lax.fori_loop(0, 391, _p1, 0)
    plsc.subcore_barrier()

    def _head_pass(hd, hcarry):
        def _zagg(j, carry):
            g = j * 16 + s

            @pl.when(g < _NCH)
            def _():
                pltpu.sync_copy(z16, agg_sp.at[pl.ds(g * 128, 128)])
            return carry

        lax.fori_loop(0, 25, _zagg, 0)
        plsc.subcore_barrier()

        hoff = (b * _HEADS + hd) * _N
        eoff = (b * _HEADS + hd) * _E

        def _scale(ewt_, hpbuf_):
            def _wm(e_i, carry):
                wv = plsc.load_gather(ewt_, [jnp.full((16,), e_i, jnp.int32)])
                hpbuf_[e_i, :] = hpbuf_[e_i, :] * wv
                return carry

            lax.fori_loop(0, 128, _wm, 0, unroll=8)

        def _p2(q, carry):
            gA = (2 * q) * 16 + s
            gB = (2 * q + 1) * 16 + s

            @pl.when(gA < _ECH)
            def _():
                base = b * _E + gA * 128
                pltpu.async_copy(src_hbm.at[pl.ds(base, 128)], srcb, semi)
                pltpu.async_copy(trg_hbm.at[pl.ds(base, 128)], trgb, semi)
                pltpu.async_copy(exp_hbm.at[pl.ds(eoff + gA * 128, 128)],
                                 ewt, semi)

            @pl.when(gB < _ECH)
            def _():
                base = b * _E + gB * 128
                pltpu.async_copy(src_hbm.at[pl.ds(base, 128)], srcb2, semi2)
                pltpu.async_copy(trg_hbm.at[pl.ds(base, 128)], trgb2, semi2)
                pltpu.async_copy(exp_hbm.at[pl.ds(eoff + gB * 128, 128)],
                                 ewt2, semi2)

            @pl.when(gA < _ECH)
            def _():
                base = b * _E + gA * 128
                pltpu.make_async_copy(src_hbm.at[pl.ds(base, 128)], srcb, semi).wait()
                pltpu.make_async_copy(trg_hbm.at[pl.ds(base, 128)], trgb, semi).wait()
                pltpu.make_async_copy(exp_hbm.at[pl.ds(eoff + gA * 128, 128)],
                                      ewt, semi).wait()
                for k in range(8):
                    offb[pl.ds(k * 16, 16)] = srcb[pl.ds(k * 16, 16)] + hoff
                pltpu.async_copy(hp_hbm.at[offb], hpbuf, semg)

            @pl.when(gB < _ECH)
            def _():
                base = b * _E + gB * 128
                pltpu.make_async_copy(src_hbm.at[pl.ds(base, 128)], srcb2, semi2).wait()
                pltpu.make_async_copy(trg_hbm.at[pl.ds(base, 128)], trgb2, semi2).wait()
                pltpu.make_async_copy(exp_hbm.at[pl.ds(eoff + gB * 128, 128)],
                                      ewt2, semi2).wait()
                for k in range(8):
                    offb2[pl.ds(k * 16, 16)] = srcb2[pl.ds(k * 16, 16)] + hoff
                pltpu.async_copy(hp_hbm.at[offb2], hpbuf2, semg2)

            @pl.when(gA < _ECH)
            def _():
                pltpu.make_async_copy(hp_hbm.at[offb], hpbuf, semg).wait()
                _scale(ewt, hpbuf)
                pltpu.async_copy(hpbuf, agg_sp.at[trgb], semsc, add=True)

            @pl.when(gB < _ECH)
            def _():
                pltpu.make_async_copy(hp_hbm.at[offb2], hpbuf2, semg2).wait()
                _scale(ewt2, hpbuf2)
                pltpu.async_copy(hpbuf2, agg_sp.at[trgb2], semsc2, add=True)

            @pl.when(gA < _ECH)
            def _():
                pltpu.make_async_copy(hpbuf, agg_sp.at[trgb], semsc).wait()

            @pl.when(gB < _ECH)
            def _():
                pltpu.make_async_copy(hpbuf2, agg_sp.at[trgb2], semsc2).wait()
            return carry

        lax.fori_loop(0, 196, _p2, 0)
        plsc.subcore_barrier()

        def _fin(j, carry):
            g = j * 16 + s

            @pl.when(g < _NFULL)
            def _():
                nb = g * 128
                pltpu.sync_copy(agg_sp.at[pl.ds(nb, 128)], abuf)
                pltpu.sync_copy(denom_sp.at[pl.ds(nb, 128)], dbuf)
                hsel = jnp.full((16,), hd, jnp.int32)

                def _ndiv(i, carry):
                    dv = plsc.load_gather(dbuf, [jnp.full((16,), i, jnp.int32), hsel])
                    abuf[i, :] = abuf[i, :] / (dv + 1e-16)
                    return carry

                lax.fori_loop(0, 128, _ndiv, 0, unroll=8)
                pltpu.sync_copy(
                    abuf, out_hbm.at[pl.ds((b * _HEADS + hd) * _N + nb, 128)])
            return carry

        lax.fori_loop(0, 25, _fin, 0)

        @pl.when(s == _TAIL_TILE)
        def _():
            nb = _NFULL * 128
            pltpu.sync_copy(agg_sp.at[pl.ds(nb, _TAIL)], abuf.at[pl.ds(0, _TAIL)])
            pltpu.sync_copy(denom_sp.at[pl.ds(nb, _TAIL)], dbuf.at[pl.ds(0, _TAIL)])
            hsel = jnp.full((16,), hd, jnp.int32)

            def _ndiv2(i, carry):
                dv = plsc.load_gather(dbuf, [jnp.full((16,), i, jnp.int32), hsel])
                abuf[i, :] = abuf[i, :] / (dv + 1e-16)
                return carry

            lax.fori_loop(0, _TAIL, _ndiv2, 0, unroll=8)
            pltpu.sync_copy(
                abuf.at[pl.ds(0, _TAIL)],
                out_hbm.at[pl.ds((b * _HEADS + hd) * _N + nb, _TAIL)])

        plsc.subcore_barrier()
        return hcarry

    lax.fori_loop(0, _HEADS, _head_pass, 0)


def kernel(h, adjs, W, attn_src, attn_trg):
    src = adjs[:, 0, :].astype(jnp.int32).reshape(_B * _E)
    trg = adjs[:, 1, :].astype(jnp.int32).reshape(_B * _E)
    hp4, asrc, atrg, mpair = _proj(h, W, attn_src, attn_trg)
    out_flat, _ = _sc_gat(
        hp4.reshape(_B * _HEADS * _N, _OUT_F),
        asrc.reshape(_B * _N, 16),
        atrg.reshape(_B * _N, 16),
        src, trg,
        mpair[:, :, 0, :16].reshape(_B * 32),
    )
    return out_flat.reshape(_B, _HEADS, _N, _OUT_F)


# speedup vs baseline: 83.5790x; 1.0749x over previous
import functools

import jax
import jax.numpy as jnp
from jax import lax
from jax.experimental import pallas as pl
from jax.experimental.pallas import tpu as pltpu
from jax.experimental.pallas import tpu_sc as plsc

_B, _N, _E = 2, 50000, 800000
_IN_F, _HEADS, _OUT_F = 128, 4, 16

_BN = 2000
_NBLK = _N // _BN


def _proj_body(h_ref, w_ref, as_ref, at_ref, hp_ref, asrc_ref, atrg_ref, m_ref):
    hb = h_ref[0]
    s_cols, t_cols = [], []
    for hd in range(_HEADS):
        wh = w_ref[hd * _OUT_F:(hd + 1) * _OUT_F, :]
        hp_h = lax.dot_general(hb, wh, (((1,), (1,)), ((), ())),
                               preferred_element_type=jnp.float32)
        hp_ref[0, hd] = hp_h
        s_cols.append(jnp.sum(hp_h * as_ref[0, hd][None, :], axis=-1, keepdims=True))
        t_cols.append(jnp.sum(hp_h * at_ref[0, hd][None, :], axis=-1, keepdims=True))
    pad = jnp.zeros((_BN, 16 - _HEADS), jnp.float32)
    asv = jnp.concatenate(s_cols + [pad], axis=1)
    atv = jnp.concatenate(t_cols + [pad], axis=1)
    asrc_ref[0] = asv
    atrg_ref[0] = atv
    cur = jnp.stack([jnp.full((8, 128), jnp.max(jnp.concatenate(s_cols, axis=1)),
                              jnp.float32),
                     jnp.full((8, 128), jnp.max(jnp.concatenate(t_cols, axis=1)),
                              jnp.float32)])

    @pl.when(pl.program_id(1) == 0)
    def _():
        m_ref[0] = cur

    @pl.when(pl.program_id(1) != 0)
    def _():
        m_ref[0] = jnp.maximum(m_ref[0], cur)


_proj = pl.pallas_call(
    _proj_body,
    grid=(_B, _NBLK),
    in_specs=[
        pl.BlockSpec((1, _BN, _IN_F), lambda b, n: (b, n, 0)),
        pl.BlockSpec((_HEADS * _OUT_F, _IN_F), lambda b, n: (0, 0)),
        pl.BlockSpec((1, _HEADS, _OUT_F), lambda b, n: (0, 0, 0)),
        pl.BlockSpec((1, _HEADS, _OUT_F), lambda b, n: (0, 0, 0)),
    ],
    out_specs=[
        pl.BlockSpec((1, _HEADS, _BN, _OUT_F), lambda b, n: (b, 0, n, 0)),
        pl.BlockSpec((1, _BN, 16), lambda b, n: (b, n, 0)),
        pl.BlockSpec((1, _BN, 16), lambda b, n: (b, n, 0)),
        pl.BlockSpec((1, 2, 8, 128), lambda b, n: (b, 0, 0, 0)),
    ],
    out_shape=[
        jax.ShapeDtypeStruct((_B, _HEADS, _N, _OUT_F), jnp.float32),
        jax.ShapeDtypeStruct((_B, _N, 16), jnp.float32),
        jax.ShapeDtypeStruct((_B, _N, 16), jnp.float32),
        jax.ShapeDtypeStruct((_B, 2, 8, 128), jnp.float32),
    ],
)

_NPAD = 50048
_ECH = _E // 128
_NCH = _NPAD // 128
_NFULL = _N // 128
_TAIL = _N - _NFULL * 128
_TAIL_TILE = _NFULL % 16

_mesh = plsc.VectorSubcoreMesh(core_axis_name="c", subcore_axis_name="s")


@functools.partial(
    pl.kernel,
    out_type=(
        jax.ShapeDtypeStruct((_B * _HEADS * _N, _OUT_F), jnp.float32),
        jax.ShapeDtypeStruct((_B * _HEADS * _E,), jnp.float32),
    ),
    mesh=_mesh,
    compiler_params=pltpu.CompilerParams(needs_layout_passes=False,
                                         use_tc_tiling_on_sc=False),
    scratch_types=[
        pltpu.VMEM_SHARED((_NPAD, 16), jnp.float32),
        pltpu.VMEM_SHARED((_NPAD, _OUT_F), jnp.float32),
        pltpu.VMEM((128,), jnp.int32),
        pltpu.VMEM((128,), jnp.int32),
        pltpu.VMEM((128,), jnp.int32),
        pltpu.VMEM((128, 16), jnp.float32),
        pltpu.VMEM((128, 16), jnp.float32),
        pltpu.VMEM((128, 16), jnp.float32),
        pltpu.VMEM((_HEADS, 128), jnp.float32),
        pltpu.VMEM((128,), jnp.float32),
        pltpu.VMEM((128, _OUT_F), jnp.float32),
        pltpu.VMEM((128, _OUT_F), jnp.float32),
        pltpu.VMEM((128, 16), jnp.float32),
        pltpu.VMEM((32,), jnp.float32),
        pltpu.VMEM((128, 16), jnp.float32),
        pltpu.VMEM((128,), jnp.int32),
        pltpu.SemaphoreType.DMA,
        pltpu.SemaphoreType.DMA,
        pltpu.SemaphoreType.DMA,
        pltpu.VMEM((128,), jnp.int32),
        pltpu.VMEM((128,), jnp.int32),
        pltpu.VMEM((128,), jnp.float32),
        pltpu.VMEM((128, _OUT_F), jnp.float32),
        pltpu.SemaphoreType.DMA,
        pltpu.SemaphoreType.DMA,
        pltpu.SemaphoreType.DMA,
        pltpu.VMEM((128,), jnp.int32),
        pltpu.VMEM((128,), jnp.int32),
        pltpu.VMEM((128, 16), jnp.float32),
        pltpu.VMEM((128, 16), jnp.float32),
        pltpu.VMEM((128, 16), jnp.float32),
        pltpu.VMEM((_HEADS, 128), jnp.float32),
    ],
)
def _sc_gat(hp_hbm, asrc_hbm, atrg_hbm, src_hbm, trg_hbm, m_hbm,
            out_hbm, exp_hbm, denom_sp, agg_sp, srcb, trgb, offb,
            sbuf, tbuf, ebuf, ehead, ewt, hpbuf, abuf, dbuf, mbuf, z16,
            offb2, semi, semg, semsc,
            srcb2, trgb2, ewt2, hpbuf2, semi2, semg2, semsc2,
            offb3, offb4, sbuf2, tbuf2, ebuf2, ehead2):
    b = lax.axis_index("c")
    s = lax.axis_index("s")
    iota = lax.iota(jnp.int32, 16)
    r0 = iota >> 2
    c0 = iota & 3
    zv = jnp.zeros((16,), jnp.float32)

    def _zfill(i, carry):
        z16[i, :] = zv
        ebuf[i, :] = zv
        ebuf2[i, :] = zv
        return carry

    lax.fori_loop(0, 128, _zfill, 0)

    pltpu.sync_copy(m_hbm.at[pl.ds(b * 32, 32)], mbuf)
    msum = mbuf[pl.ds(0, 16)] + mbuf[pl.ds(16, 16)]
    mvec = jnp.maximum(msum, 0.2 * msum)

    def _zden(j, carry):
        g = j * 16 + s

        @pl.when(g < _NCH)
        def _():
            pltpu.sync_copy(z16, denom_sp.at[pl.ds(g * 128, 128)])
        return carry

    lax.fori_loop(0, 25, _zden, 0)
    plsc.subcore_barrier()

    noff = b * _N

    def _p1compute(sbuf_, tbuf_, ebuf_):
        for k in range(32):
            rows = r0 + 4 * k
            sv = plsc.load_gather(sbuf_, [rows, c0])
            tv = plsc.load_gather(tbuf_, [rows, c0])
            x = sv + tv
            ev = jnp.exp(jnp.maximum(x, 0.2 * x) - mvec)
            plsc.store_scatter(ebuf_, [rows, c0], ev)

    def _ehead_build(ebuf_, ehead_):
        for hd in range(_HEADS):
            hsel = jnp.full((16,), hd, jnp.int32)
            for k in range(8):
                ehead_[hd, pl.ds(k * 16, 16)] = plsc.load_gather(
                    ebuf_, [iota + 16 * k, hsel])

    def _p1(q, carry):
        gA = (2 * q) * 16 + s
        gB = (2 * q + 1) * 16 + s

        @pl.when(gA < _ECH)
        def _():
            base = b * _E + gA * 128
            pltpu.async_copy(src_hbm.at[pl.ds(base, 128)], srcb, semi)
            pltpu.async_copy(trg_hbm.at[pl.ds(base, 128)], trgb, semi)

        @pl.when(gB < _ECH)
        def _():
            base = b * _E + gB * 128
            pltpu.async_copy(src_hbm.at[pl.ds(base, 128)], srcb2, semi2)
            pltpu.async_copy(trg_hbm.at[pl.ds(base, 128)], trgb2, semi2)

        @pl.when(gA < _ECH)
        def _():
            base = b * _E + gA * 128
            pltpu.make_async_copy(src_hbm.at[pl.ds(base, 128)], srcb, semi).wait()
            pltpu.make_async_copy(trg_hbm.at[pl.ds(base, 128)], trgb, semi).wait()
            for k in range(8):
                offb[pl.ds(k * 16, 16)] = srcb[pl.ds(k * 16, 16)] + noff
                offb2[pl.ds(k * 16, 16)] = trgb[pl.ds(k * 16, 16)] + noff
            pltpu.async_copy(asrc_hbm.at[offb], sbuf, semg)
            pltpu.async_copy(atrg_hbm.at[offb2], tbuf, semg)

        @pl.when(gB < _ECH)
        def _():
            base = b * _E + gB * 128
            pltpu.make_async_copy(src_hbm.at[pl.ds(base, 128)], srcb2, semi2).wait()
            pltpu.make_async_copy(trg_hbm.at[pl.ds(base, 128)], trgb2, semi2).wait()
            for k in range(8):
                offb3[pl.ds(k * 16, 16)] = srcb2[pl.ds(k * 16, 16)] + noff
                offb4[pl.ds(k * 16, 16)] = trgb2[pl.ds(k * 16, 16)] + noff
            pltpu.async_copy(asrc_hbm.at[offb3], sbuf2, semg2)
            pltpu.async_copy(atrg_hbm.at[offb4], tbuf2, semg2)

        @pl.when(gA < _ECH)
        def _():
            pltpu.make_async_copy(asrc_hbm.at[offb], sbuf, semg).wait()
            pltpu.make_async_copy(atrg_hbm.at[offb2], tbuf, semg).wait()
            _p1compute(sbuf, tbuf, ebuf)
            pltpu.async_copy(ebuf, denom_sp.at[trgb], semsc, add=True)
            _ehead_build(ebuf, ehead)
            for hd in range(_HEADS):
                pltpu.async_copy(
                    ehead.at[hd],
                    exp_hbm.at[pl.ds((b * _HEADS + hd) * _E + gA * 128, 128)],
                    semg)

        @pl.when(gB < _ECH)
        def _():
            pltpu.make_async_copy(asrc_hbm.at[offb3], sbuf2, semg2).wait()
            pltpu.make_async_copy(atrg_hbm.at[offb4], tbuf2, semg2).wait()
            _p1compute(sbuf2, tbuf2, ebuf2)
            pltpu.async_copy(ebuf2, denom_sp.at[trgb2], semsc2, add=True)
            _ehead_build(ebuf2, ehead2)
            for hd in range(_HEADS):
                pltpu.async_copy(
                    ehead2.at[hd],
                    exp_hbm.at[pl.ds((b * _HEADS + hd) * _E + gB * 128, 128)],
                    semg2)

        @pl.when(gA < _ECH)
        def _():
            pltpu.make_async_copy(ebuf, denom_sp.at[trgb], semsc).wait()
            for hd in range(_HEADS):
                pltpu.make_async_copy(
                    ehead.at[hd],
                    exp_hbm.at[pl.ds((b * _HEADS + hd) * _E + gA * 128, 128)],
                    semg).wait()

        @pl.when(gB < _ECH)
        def _():
            pltpu.make_async_copy(ebuf2, denom_sp.at[trgb2], semsc2).wait()
            for hd in range(_HEADS):
                pltpu.make_async_copy(
                    ehead2.at[hd],
                    exp_hbm.at[pl.ds((b * _HEADS + hd) * _E + gB * 128, 128)],
                    semg2).wait()
        return carry

    lax.fori_loop(0, 196, _p1, 0)
    plsc.subcore_barrier()

    def _head_pass(hd, hcarry):
        def _zagg(j, carry):
            g = j * 16 + s

            @pl.when(g < _NCH)
            def _():
                pltpu.sync_copy(z16, agg_sp.at[pl.ds(g * 128, 128)])
            return carry

        lax.fori_loop(0, 25, _zagg, 0)
        plsc.subcore_barrier()

        hoff = (b * _HEADS + hd) * _N
        eoff = (b * _HEADS + hd) * _E

        def _scale(ewt_, hpbuf_):
            def _wm(e_i, carry):
                wv = plsc.load_gather(ewt_, [jnp.full((16,), e_i, jnp.int32)])
                hpbuf_[e_i, :] = hpbuf_[e_i, :] * wv
                return carry

            lax.fori_loop(0, 128, _wm, 0, unroll=8)

        def _p2(q, carry):
            gA = (2 * q) * 16 + s
            gB = (2 * q + 1) * 16 + s

            @pl.when(gA < _ECH)
            def _():
                base = b * _E + gA * 128
                pltpu.async_copy(src_hbm.at[pl.ds(base, 128)], srcb, semi)
                pltpu.async_copy(trg_hbm.at[pl.ds(base, 128)], trgb, semi)
                pltpu.async_copy(exp_hbm.at[pl.ds(eoff + gA * 128, 128)],
                                 ewt, semi)

            @pl.when(gB < _ECH)
            def _():
                base = b * _E + gB * 128
                pltpu.async_copy(src_hbm.at[pl.ds(base, 128)], srcb2, semi2)
                pltpu.async_copy(trg_hbm.at[pl.ds(base, 128)], trgb2, semi2)
                pltpu.async_copy(exp_hbm.at[pl.ds(eoff + gB * 128, 128)],
                                 ewt2, semi2)

            @pl.when(gA < _ECH)
            def _():
                base = b * _E + gA * 128
                pltpu.make_async_copy(src_hbm.at[pl.ds(base, 128)], srcb, semi).wait()
                pltpu.make_async_copy(trg_hbm.at[pl.ds(base, 128)], trgb, semi).wait()
                pltpu.make_async_copy(exp_hbm.at[pl.ds(eoff + gA * 128, 128)],
                                      ewt, semi).wait()
                for k in range(8):
                    offb[pl.ds(k * 16, 16)] = srcb[pl.ds(k * 16, 16)] + hoff
                pltpu.async_copy(hp_hbm.at[offb], hpbuf, semg)

            @pl.when(gB < _ECH)
            def _():
                base = b * _E + gB * 128
                pltpu.make_async_copy(src_hbm.at[pl.ds(base, 128)], srcb2, semi2).wait()
                pltpu.make_async_copy(trg_hbm.at[pl.ds(base, 128)], trgb2, semi2).wait()
                pltpu.make_async_copy(exp_hbm.at[pl.ds(eoff + gB * 128, 128)],
                                      ewt2, semi2).wait()
                for k in range(8):
                    offb2[pl.ds(k * 16, 16)] = srcb2[pl.ds(k * 16, 16)] + hoff
                pltpu.async_copy(hp_hbm.at[offb2], hpbuf2, semg2)

            @pl.when(gA < _ECH)
            def _():
                pltpu.make_async_copy(hp_hbm.at[offb], hpbuf, semg).wait()
                _scale(ewt, hpbuf)
                pltpu.async_copy(hpbuf, agg_sp.at[trgb], semsc, add=True)

            @pl.when(gB < _ECH)
            def _():
                pltpu.make_async_copy(hp_hbm.at[offb2], hpbuf2, semg2).wait()
                _scale(ewt2, hpbuf2)
                pltpu.async_copy(hpbuf2, agg_sp.at[trgb2], semsc2, add=True)

            @pl.when(gA < _ECH)
            def _():
                pltpu.make_async_copy(hpbuf, agg_sp.at[trgb], semsc).wait()

            @pl.when(gB < _ECH)
            def _():
                pltpu.make_async_copy(hpbuf2, agg_sp.at[trgb2], semsc2).wait()
            return carry

        lax.fori_loop(0, 196, _p2, 0)
        plsc.subcore_barrier()

        def _fin(j, carry):
            g = j * 16 + s

            @pl.when(g < _NFULL)
            def _():
                nb = g * 128
                pltpu.sync_copy(agg_sp.at[pl.ds(nb, 128)], abuf)
                pltpu.sync_copy(denom_sp.at[pl.ds(nb, 128)], dbuf)
                hsel = jnp.full((16,), hd, jnp.int32)

                def _ndiv(i, carry):
                    dv = plsc.load_gather(dbuf, [jnp.full((16,), i, jnp.int32), hsel])
                    abuf[i, :] = abuf[i, :] / (dv + 1e-16)
                    return carry

                lax.fori_loop(0, 128, _ndiv, 0, unroll=8)
                pltpu.sync_copy(
                    abuf, out_hbm.at[pl.ds((b * _HEADS + hd) * _N + nb, 128)])
            return carry

        lax.fori_loop(0, 25, _fin, 0)

        @pl.when(s == _TAIL_TILE)
        def _():
            nb = _NFULL * 128
            pltpu.sync_copy(agg_sp.at[pl.ds(nb, _TAIL)], abuf.at[pl.ds(0, _TAIL)])
            pltpu.sync_copy(denom_sp.at[pl.ds(nb, _TAIL)], dbuf.at[pl.ds(0, _TAIL)])
            hsel = jnp.full((16,), hd, jnp.int32)

            def _ndiv2(i, carry):
                dv = plsc.load_gather(dbuf, [jnp.full((16,), i, jnp.int32), hsel])
                abuf[i, :] = abuf[i, :] / (dv + 1e-16)
                return carry

            lax.fori_loop(0, _TAIL, _ndiv2, 0, unroll=8)
            pltpu.sync_copy(
                abuf.at[pl.ds(0, _TAIL)],
                out_hbm.at[pl.ds((b * _HEADS + hd) * _N + nb, _TAIL)])

        plsc.subcore_barrier()
        return hcarry

    lax.fori_loop(0, _HEADS, _head_pass, 0)


def kernel(h, adjs, W, attn_src, attn_trg):
    src = adjs[:, 0, :].astype(jnp.int32).reshape(_B * _E)
    trg = adjs[:, 1, :].astype(jnp.int32).reshape(_B * _E)
    hp4, asrc, atrg, mpair = _proj(h, W, attn_src, attn_trg)
    out_flat, _ = _sc_gat(
        hp4.reshape(_B * _HEADS * _N, _OUT_F),
        asrc.reshape(_B * _N, 16),
        atrg.reshape(_B * _N, 16),
        src, trg,
        mpair[:, :, 0, :16].reshape(_B * 32),
    )
    return out_flat.reshape(_B, _HEADS, _N, _OUT_F)


# speedup vs baseline: 97.9086x; 1.1714x over previous
import functools

import jax
import jax.numpy as jnp
from jax import lax
from jax.experimental import pallas as pl
from jax.experimental.pallas import tpu as pltpu
from jax.experimental.pallas import tpu_sc as plsc

_B, _N, _E = 2, 50000, 800000
_IN_F, _HEADS, _OUT_F = 128, 4, 16

_BN = 2000
_NBLK = _N // _BN


def _proj_body(h_ref, w_ref, as_ref, at_ref, hp_ref, asrc_ref, atrg_ref, m_ref):
    hb = h_ref[0]
    s_cols, t_cols = [], []
    for hd in range(_HEADS):
        wh = w_ref[hd * _OUT_F:(hd + 1) * _OUT_F, :]
        hp_h = lax.dot_general(hb, wh, (((1,), (1,)), ((), ())),
                               preferred_element_type=jnp.float32)
        hp_ref[0, hd] = hp_h
        s_cols.append(jnp.sum(hp_h * as_ref[0, hd][None, :], axis=-1, keepdims=True))
        t_cols.append(jnp.sum(hp_h * at_ref[0, hd][None, :], axis=-1, keepdims=True))
    pad = jnp.zeros((_BN, 16 - _HEADS), jnp.float32)
    asv = jnp.concatenate(s_cols + [pad], axis=1)
    atv = jnp.concatenate(t_cols + [pad], axis=1)
    asrc_ref[0] = asv
    atrg_ref[0] = atv
    cur = jnp.stack([jnp.full((8, 128), jnp.max(jnp.concatenate(s_cols, axis=1)),
                              jnp.float32),
                     jnp.full((8, 128), jnp.max(jnp.concatenate(t_cols, axis=1)),
                              jnp.float32)])

    @pl.when(pl.program_id(1) == 0)
    def _():
        m_ref[0] = cur

    @pl.when(pl.program_id(1) != 0)
    def _():
        m_ref[0] = jnp.maximum(m_ref[0], cur)


_proj = pl.pallas_call(
    _proj_body,
    grid=(_B, _NBLK),
    in_specs=[
        pl.BlockSpec((1, _BN, _IN_F), lambda b, n: (b, n, 0)),
        pl.BlockSpec((_HEADS * _OUT_F, _IN_F), lambda b, n: (0, 0)),
        pl.BlockSpec((1, _HEADS, _OUT_F), lambda b, n: (0, 0, 0)),
        pl.BlockSpec((1, _HEADS, _OUT_F), lambda b, n: (0, 0, 0)),
    ],
    out_specs=[
        pl.BlockSpec((1, _HEADS, _BN, _OUT_F), lambda b, n: (b, 0, n, 0)),
        pl.BlockSpec((1, _BN, 16), lambda b, n: (b, n, 0)),
        pl.BlockSpec((1, _BN, 16), lambda b, n: (b, n, 0)),
        pl.BlockSpec((1, 2, 8, 128), lambda b, n: (b, 0, 0, 0)),
    ],
    out_shape=[
        jax.ShapeDtypeStruct((_B, _HEADS, _N, _OUT_F), jnp.float32),
        jax.ShapeDtypeStruct((_B, _N, 16), jnp.float32),
        jax.ShapeDtypeStruct((_B, _N, 16), jnp.float32),
        jax.ShapeDtypeStruct((_B, 2, 8, 128), jnp.float32),
    ],
)

_NPAD = 50048
_ECH = _E // 128
_NCH = _NPAD // 128
_NFULL = _N // 128
_TAIL = _N - _NFULL * 128
_TAIL_TILE = _NFULL % 16

_mesh = plsc.VectorSubcoreMesh(core_axis_name="c", subcore_axis_name="s")


@functools.partial(
    pl.kernel,
    out_type=(
        jax.ShapeDtypeStruct((_B * _HEADS * _N, _OUT_F), jnp.float32),
        jax.ShapeDtypeStruct((_B * _HEADS * _E,), jnp.float32),
    ),
    mesh=_mesh,
    compiler_params=pltpu.CompilerParams(needs_layout_passes=False,
                                         use_tc_tiling_on_sc=False),
    scratch_types=[
        pltpu.VMEM_SHARED((_NPAD, 16), jnp.float32),
        pltpu.VMEM_SHARED((_NPAD, _OUT_F), jnp.float32),
        pltpu.VMEM((128,), jnp.int32),
        pltpu.VMEM((128,), jnp.int32),
        pltpu.VMEM((128,), jnp.int32),
        pltpu.VMEM((128, 16), jnp.float32),
        pltpu.VMEM((128, 16), jnp.float32),
        pltpu.VMEM((128, 16), jnp.float32),
        pltpu.VMEM((_HEADS, 128), jnp.float32),
        pltpu.VMEM((128,), jnp.float32),
        pltpu.VMEM((128, _OUT_F), jnp.float32),
        pltpu.VMEM((128, _OUT_F), jnp.float32),
        pltpu.VMEM((128, 16), jnp.float32),
        pltpu.VMEM((32,), jnp.float32),
        pltpu.VMEM((128, 16), jnp.float32),
        pltpu.VMEM((128,), jnp.int32),
        pltpu.SemaphoreType.DMA,
        pltpu.SemaphoreType.DMA,
        pltpu.SemaphoreType.DMA,
        pltpu.VMEM((128,), jnp.int32),
        pltpu.VMEM((128,), jnp.int32),
        pltpu.VMEM((128,), jnp.float32),
        pltpu.VMEM((128, _OUT_F), jnp.float32),
        pltpu.SemaphoreType.DMA,
        pltpu.SemaphoreType.DMA,
        pltpu.SemaphoreType.DMA,
        pltpu.VMEM((128,), jnp.int32),
        pltpu.VMEM((128,), jnp.int32),
        pltpu.VMEM((128, 16), jnp.float32),
        pltpu.VMEM((128, 16), jnp.float32),
        pltpu.VMEM((128, 16), jnp.float32),
        pltpu.VMEM((_HEADS, 128), jnp.float32),
        pltpu.VMEM((128,), jnp.int32),
        pltpu.VMEM((128,), jnp.int32),
        pltpu.VMEM((128,), jnp.float32),
        pltpu.VMEM((128, _OUT_F), jnp.float32),
        pltpu.VMEM((128,), jnp.int32),
        pltpu.VMEM((128,), jnp.int32),
        pltpu.VMEM((128,), jnp.float32),
        pltpu.VMEM((128, _OUT_F), jnp.float32),
        pltpu.SemaphoreType.DMA,
        pltpu.SemaphoreType.DMA,
        pltpu.SemaphoreType.DMA,
        pltpu.SemaphoreType.DMA,
        pltpu.SemaphoreType.DMA,
        pltpu.SemaphoreType.DMA,
    ],
)
def _sc_gat(hp_hbm, asrc_hbm, atrg_hbm, src_hbm, trg_hbm, m_hbm,
            out_hbm, exp_hbm, denom_sp, agg_sp, srcb, trgb, offb,
            sbuf, tbuf, ebuf, ehead, ewt, hpbuf, abuf, dbuf, mbuf, z16,
            offb2, semi, semg, semsc,
            srcb2, trgb2, ewt2, hpbuf2, semi2, semg2, semsc2,
            offb3, offb4, sbuf2, tbuf2, ebuf2, ehead2,
            srcb3, trgb3, ewt3, hpbuf3, srcb4, trgb4, ewt4, hpbuf4,
            semi3, semg3, semsc3, semi4, semg4, semsc4):
    b = lax.axis_index("c")
    s = lax.axis_index("s")
    iota = lax.iota(jnp.int32, 16)
    r0 = iota >> 2
    c0 = iota & 3
    zv = jnp.zeros((16,), jnp.float32)

    def _zfill(i, carry):
        z16[i, :] = zv
        ebuf[i, :] = zv
        ebuf2[i, :] = zv
        return carry

    lax.fori_loop(0, 128, _zfill, 0)

    pltpu.sync_copy(m_hbm.at[pl.ds(b * 32, 32)], mbuf)
    msum = mbuf[pl.ds(0, 16)] + mbuf[pl.ds(16, 16)]
    mvec = jnp.maximum(msum, 0.2 * msum)

    def _zden(j, carry):
        g = j * 16 + s

        @pl.when(g < _NCH)
        def _():
            pltpu.sync_copy(z16, denom_sp.at[pl.ds(g * 128, 128)])
        return carry

    lax.fori_loop(0, 25, _zden, 0)
    plsc.subcore_barrier()

    noff = b * _N

    def _p1compute(sbuf_, tbuf_, ebuf_):
        for k in range(32):
            rows = r0 + 4 * k
            sv = plsc.load_gather(sbuf_, [rows, c0])
            tv = plsc.load_gather(tbuf_, [rows, c0])
            x = sv + tv
            ev = jnp.exp(jnp.maximum(x, 0.2 * x) - mvec)
            plsc.store_scatter(ebuf_, [rows, c0], ev)

    def _ehead_build(ebuf_, ehead_):
        for hd in range(_HEADS):
            hsel = jnp.full((16,), hd, jnp.int32)
            for k in range(8):
                ehead_[hd, pl.ds(k * 16, 16)] = plsc.load_gather(
                    ebuf_, [iota + 16 * k, hsel])

    def _p1(q, carry):
        gA = (2 * q) * 16 + s
        gB = (2 * q + 1) * 16 + s

        @pl.when(gA < _ECH)
        def _():
            base = b * _E + gA * 128
            pltpu.async_copy(src_hbm.at[pl.ds(base, 128)], srcb, semi)
            pltpu.async_copy(trg_hbm.at[pl.ds(base, 128)], trgb, semi)

        @pl.when(gB < _ECH)
        def _():
            base = b * _E + gB * 128
            pltpu.async_copy(src_hbm.at[pl.ds(base, 128)], srcb2, semi2)
            pltpu.async_copy(trg_hbm.at[pl.ds(base, 128)], trgb2, semi2)

        @pl.when(gA < _ECH)
        def _():
            base = b * _E + gA * 128
            pltpu.make_async_copy(src_hbm.at[pl.ds(base, 128)], srcb, semi).wait()
            pltpu.make_async_copy(trg_hbm.at[pl.ds(base, 128)], trgb, semi).wait()
            for k in range(8):
                offb[pl.ds(k * 16, 16)] = srcb[pl.ds(k * 16, 16)] + noff
                offb2[pl.ds(k * 16, 16)] = trgb[pl.ds(k * 16, 16)] + noff
            pltpu.async_copy(asrc_hbm.at[offb], sbuf, semg)
            pltpu.async_copy(atrg_hbm.at[offb2], tbuf, semg)

        @pl.when(gB < _ECH)
        def _():
            base = b * _E + gB * 128
            pltpu.make_async_copy(src_hbm.at[pl.ds(base, 128)], srcb2, semi2).wait()
            pltpu.make_async_copy(trg_hbm.at[pl.ds(base, 128)], trgb2, semi2).wait()
            for k in range(8):
                offb3[pl.ds(k * 16, 16)] = srcb2[pl.ds(k * 16, 16)] + noff
                offb4[pl.ds(k * 16, 16)] = trgb2[pl.ds(k * 16, 16)] + noff
            pltpu.async_copy(asrc_hbm.at[offb3], sbuf2, semg2)
            pltpu.async_copy(atrg_hbm.at[offb4], tbuf2, semg2)

        @pl.when(gA < _ECH)
        def _():
            pltpu.make_async_copy(asrc_hbm.at[offb], sbuf, semg).wait()
            pltpu.make_async_copy(atrg_hbm.at[offb2], tbuf, semg).wait()
            _p1compute(sbuf, tbuf, ebuf)
            pltpu.async_copy(ebuf, denom_sp.at[trgb], semsc, add=True)
            _ehead_build(ebuf, ehead)
            for hd in range(_HEADS):
                pltpu.async_copy(
                    ehead.at[hd],
                    exp_hbm.at[pl.ds((b * _HEADS + hd) * _E + gA * 128, 128)],
                    semg)

        @pl.when(gB < _ECH)
        def _():
            pltpu.make_async_copy(asrc_hbm.at[offb3], sbuf2, semg2).wait()
            pltpu.make_async_copy(atrg_hbm.at[offb4], tbuf2, semg2).wait()
            _p1compute(sbuf2, tbuf2, ebuf2)
            pltpu.async_copy(ebuf2, denom_sp.at[trgb2], semsc2, add=True)
            _ehead_build(ebuf2, ehead2)
            for hd in range(_HEADS):
                pltpu.async_copy(
                    ehead2.at[hd],
                    exp_hbm.at[pl.ds((b * _HEADS + hd) * _E + gB * 128, 128)],
                    semg2)

        @pl.when(gA < _ECH)
        def _():
            pltpu.make_async_copy(ebuf, denom_sp.at[trgb], semsc).wait()
            for hd in range(_HEADS):
                pltpu.make_async_copy(
                    ehead.at[hd],
                    exp_hbm.at[pl.ds((b * _HEADS + hd) * _E + gA * 128, 128)],
                    semg).wait()

        @pl.when(gB < _ECH)
        def _():
            pltpu.make_async_copy(ebuf2, denom_sp.at[trgb2], semsc2).wait()
            for hd in range(_HEADS):
                pltpu.make_async_copy(
                    ehead2.at[hd],
                    exp_hbm.at[pl.ds((b * _HEADS + hd) * _E + gB * 128, 128)],
                    semg2).wait()
        return carry

    lax.fori_loop(0, 196, _p1, 0)
    plsc.subcore_barrier()

    def _head_pass(hd, hcarry):
        def _zagg(j, carry):
            g = j * 16 + s

            @pl.when(g < _NCH)
            def _():
                pltpu.sync_copy(z16, agg_sp.at[pl.ds(g * 128, 128)])
            return carry

        lax.fori_loop(0, 25, _zagg, 0)
        plsc.subcore_barrier()

        hoff = (b * _HEADS + hd) * _N
        eoff = (b * _HEADS + hd) * _E

        def _scale(ewt_, hpbuf_):
            def _wm(e_i, carry):
                wv = plsc.load_gather(ewt_, [jnp.full((16,), e_i, jnp.int32)])
                hpbuf_[e_i, :] = hpbuf_[e_i, :] * wv
                return carry

            lax.fori_loop(0, 128, _wm, 0, unroll=8)

        p2s = [(srcb, trgb, offb, ewt, hpbuf, semi, semg, semsc),
               (srcb2, trgb2, offb2, ewt2, hpbuf2, semi2, semg2, semsc2),
               (srcb3, trgb3, offb3, ewt3, hpbuf3, semi3, semg3, semsc3),
               (srcb4, trgb4, offb4, ewt4, hpbuf4, semi4, semg4, semsc4)]

        def _p2(q, carry):
            gs = [(4 * q + t) * 16 + s for t in range(4)]
            for g_, (sb, tb, ob, ew, hb, si, sg, sc) in zip(gs, p2s):
                @pl.when(g_ < _ECH)
                def _(g_=g_, sb=sb, tb=tb, ew=ew, si=si):
                    base = b * _E + g_ * 128
                    pltpu.async_copy(src_hbm.at[pl.ds(base, 128)], sb, si)
                    pltpu.async_copy(trg_hbm.at[pl.ds(base, 128)], tb, si)
                    pltpu.async_copy(exp_hbm.at[pl.ds(eoff + g_ * 128, 128)],
                                     ew, si)
            for g_, (sb, tb, ob, ew, hb, si, sg, sc) in zip(gs, p2s):
                @pl.when(g_ < _ECH)
                def _(g_=g_, sb=sb, tb=tb, ob=ob, ew=ew, hb=hb, si=si, sg=sg):
                    base = b * _E + g_ * 128
                    pltpu.make_async_copy(
                        src_hbm.at[pl.ds(base, 128)], sb, si).wait()
                    pltpu.make_async_copy(
                        trg_hbm.at[pl.ds(base, 128)], tb, si).wait()
                    pltpu.make_async_copy(
                        exp_hbm.at[pl.ds(eoff + g_ * 128, 128)], ew, si).wait()
                    for k in range(8):
                        ob[pl.ds(k * 16, 16)] = sb[pl.ds(k * 16, 16)] + hoff
                    pltpu.async_copy(hp_hbm.at[ob], hb, sg)
            for g_, (sb, tb, ob, ew, hb, si, sg, sc) in zip(gs, p2s):
                @pl.when(g_ < _ECH)
                def _(g_=g_, tb=tb, ob=ob, ew=ew, hb=hb, sg=sg, sc=sc):
                    pltpu.make_async_copy(hp_hbm.at[ob], hb, sg).wait()
                    _scale(ew, hb)
                    pltpu.async_copy(hb, agg_sp.at[tb], sc, add=True)
            for g_, (sb, tb, ob, ew, hb, si, sg, sc) in zip(gs, p2s):
                @pl.when(g_ < _ECH)
                def _(g_=g_, tb=tb, hb=hb, sc=sc):
                    pltpu.make_async_copy(hb, agg_sp.at[tb], sc).wait()
            return carry

        lax.fori_loop(0, 98, _p2, 0)
        plsc.subcore_barrier()

        def _fin(j, carry):
            g = j * 16 + s

            @pl.when(g < _NFULL)
            def _():
                nb = g * 128
                pltpu.sync_copy(agg_sp.at[pl.ds(nb, 128)], abuf)
                pltpu.sync_copy(denom_sp.at[pl.ds(nb, 128)], dbuf)
                hsel = jnp.full((16,), hd, jnp.int32)

                def _ndiv(i, carry):
                    dv = plsc.load_gather(dbuf, [jnp.full((16,), i, jnp.int32), hsel])
                    abuf[i, :] = abuf[i, :] / (dv + 1e-16)
                    return carry

                lax.fori_loop(0, 128, _ndiv, 0, unroll=8)
                pltpu.sync_copy(
                    abuf, out_hbm.at[pl.ds((b * _HEADS + hd) * _N + nb, 128)])
            return carry

        lax.fori_loop(0, 25, _fin, 0)

        @pl.when(s == _TAIL_TILE)
        def _():
            nb = _NFULL * 128
            pltpu.sync_copy(agg_sp.at[pl.ds(nb, _TAIL)], abuf.at[pl.ds(0, _TAIL)])
            pltpu.sync_copy(denom_sp.at[pl.ds(nb, _TAIL)], dbuf.at[pl.ds(0, _TAIL)])
            hsel = jnp.full((16,), hd, jnp.int32)

            def _ndiv2(i, carry):
                dv = plsc.load_gather(dbuf, [jnp.full((16,), i, jnp.int32), hsel])
                abuf[i, :] = abuf[i, :] / (dv + 1e-16)
                return carry

            lax.fori_loop(0, _TAIL, _ndiv2, 0, unroll=8)
            pltpu.sync_copy(
                abuf.at[pl.ds(0, _TAIL)],
                out_hbm.at[pl.ds((b * _HEADS + hd) * _N + nb, _TAIL)])

        plsc.subcore_barrier()
        return hcarry

    lax.fori_loop(0, _HEADS, _head_pass, 0)


def kernel(h, adjs, W, attn_src, attn_trg):
    src = adjs[:, 0, :].astype(jnp.int32).reshape(_B * _E)
    trg = adjs[:, 1, :].astype(jnp.int32).reshape(_B * _E)
    hp4, asrc, atrg, mpair = _proj(h, W, attn_src, attn_trg)
    out_flat, _ = _sc_gat(
        hp4.reshape(_B * _HEADS * _N, _OUT_F),
        asrc.reshape(_B * _N, 16),
        atrg.reshape(_B * _N, 16),
        src, trg,
        mpair[:, :, 0, :16].reshape(_B * 32),
    )
    return out_flat.reshape(_B, _HEADS, _N, _OUT_F)


# speedup vs baseline: 100.1114x; 1.0225x over previous
import functools

import jax
import jax.numpy as jnp
from jax import lax
from jax.experimental import pallas as pl
from jax.experimental.pallas import tpu as pltpu
from jax.experimental.pallas import tpu_sc as plsc

_B, _N, _E = 2, 50000, 800000
_IN_F, _HEADS, _OUT_F = 128, 4, 16

_BN = 2000
_NBLK = _N // _BN


def _proj_body(h_ref, w_ref, as_ref, at_ref, hp_ref, asrc_ref, atrg_ref, m_ref):
    hb = h_ref[0]
    s_cols, t_cols = [], []
    for hd in range(_HEADS):
        wh = w_ref[hd * _OUT_F:(hd + 1) * _OUT_F, :]
        hp_h = lax.dot_general(hb, wh, (((1,), (1,)), ((), ())),
                               preferred_element_type=jnp.float32)
        hp_ref[0, hd] = hp_h
        s_cols.append(jnp.sum(hp_h * as_ref[0, hd][None, :], axis=-1, keepdims=True))
        t_cols.append(jnp.sum(hp_h * at_ref[0, hd][None, :], axis=-1, keepdims=True))
    pad = jnp.zeros((_BN, 16 - _HEADS), jnp.float32)
    asv = jnp.concatenate(s_cols + [pad], axis=1)
    atv = jnp.concatenate(t_cols + [pad], axis=1)
    asrc_ref[0] = asv
    atrg_ref[0] = atv
    cur = jnp.stack([jnp.full((8, 128), jnp.max(jnp.concatenate(s_cols, axis=1)),
                              jnp.float32),
                     jnp.full((8, 128), jnp.max(jnp.concatenate(t_cols, axis=1)),
                              jnp.float32)])

    @pl.when(pl.program_id(1) == 0)
    def _():
        m_ref[0] = cur

    @pl.when(pl.program_id(1) != 0)
    def _():
        m_ref[0] = jnp.maximum(m_ref[0], cur)


_proj = pl.pallas_call(
    _proj_body,
    grid=(_B, _NBLK),
    in_specs=[
        pl.BlockSpec((1, _BN, _IN_F), lambda b, n: (b, n, 0)),
        pl.BlockSpec((_HEADS * _OUT_F, _IN_F), lambda b, n: (0, 0)),
        pl.BlockSpec((1, _HEADS, _OUT_F), lambda b, n: (0, 0, 0)),
        pl.BlockSpec((1, _HEADS, _OUT_F), lambda b, n: (0, 0, 0)),
    ],
    out_specs=[
        pl.BlockSpec((1, _HEADS, _BN, _OUT_F), lambda b, n: (b, 0, n, 0)),
        pl.BlockSpec((1, _BN, 16), lambda b, n: (b, n, 0)),
        pl.BlockSpec((1, _BN, 16), lambda b, n: (b, n, 0)),
        pl.BlockSpec((1, 2, 8, 128), lambda b, n: (b, 0, 0, 0)),
    ],
    out_shape=[
        jax.ShapeDtypeStruct((_B, _HEADS, _N, _OUT_F), jnp.float32),
        jax.ShapeDtypeStruct((_B, _N, 16), jnp.float32),
        jax.ShapeDtypeStruct((_B, _N, 16), jnp.float32),
        jax.ShapeDtypeStruct((_B, 2, 8, 128), jnp.float32),
    ],
)

_NPAD = 50048
_ECH = _E // 128
_NCH = _NPAD // 128
_NFULL = _N // 128
_TAIL = _N - _NFULL * 128
_TAIL_TILE = _NFULL % 16

_mesh = plsc.VectorSubcoreMesh(core_axis_name="c", subcore_axis_name="s")


@functools.partial(
    pl.kernel,
    out_type=(
        jax.ShapeDtypeStruct((_B * _HEADS * _N, _OUT_F), jnp.float32),
        jax.ShapeDtypeStruct((_B * _ECH, _HEADS, 128), jnp.float32),
    ),
    mesh=_mesh,
    compiler_params=pltpu.CompilerParams(needs_layout_passes=False,
                                         use_tc_tiling_on_sc=False),
    scratch_types=[
        pltpu.VMEM_SHARED((_NPAD, 16), jnp.float32),
        pltpu.VMEM_SHARED((_NPAD, _OUT_F), jnp.float32),
        pltpu.VMEM((128,), jnp.int32),
        pltpu.VMEM((128,), jnp.int32),
        pltpu.VMEM((128,), jnp.int32),
        pltpu.VMEM((128, 16), jnp.float32),
        pltpu.VMEM((128, 16), jnp.float32),
        pltpu.VMEM((128, 16), jnp.float32),
        pltpu.VMEM((_HEADS, 128), jnp.float32),
        pltpu.VMEM((128,), jnp.float32),
        pltpu.VMEM((32,), jnp.float32),
        pltpu.VMEM((128,), jnp.int32),
        pltpu.SemaphoreType.DMA,
        pltpu.SemaphoreType.DMA,
        pltpu.SemaphoreType.DMA,
        pltpu.VMEM((128,), jnp.int32),
        pltpu.VMEM((128,), jnp.int32),
        pltpu.VMEM((128,), jnp.float32),
        pltpu.SemaphoreType.DMA,
        pltpu.SemaphoreType.DMA,
        pltpu.SemaphoreType.DMA,
        pltpu.VMEM((128,), jnp.int32),
        pltpu.VMEM((128,), jnp.int32),
        pltpu.VMEM((128, 16), jnp.float32),
        pltpu.VMEM((128, 16), jnp.float32),
        pltpu.VMEM((128, 16), jnp.float32),
        pltpu.VMEM((_HEADS, 128), jnp.float32),
        pltpu.VMEM((128,), jnp.int32),
        pltpu.VMEM((128,), jnp.int32),
        pltpu.VMEM((128,), jnp.float32),
        pltpu.VMEM((128,), jnp.int32),
        pltpu.VMEM((128,), jnp.int32),
        pltpu.VMEM((128,), jnp.float32),
        pltpu.SemaphoreType.DMA,
        pltpu.SemaphoreType.DMA,
        pltpu.SemaphoreType.DMA,
        pltpu.SemaphoreType.DMA,
        pltpu.SemaphoreType.DMA,
        pltpu.SemaphoreType.DMA,
        pltpu.VMEM((128,), jnp.int32),
        pltpu.VMEM((128,), jnp.int32),
        pltpu.VMEM((128,), jnp.int32),
        pltpu.VMEM((128,), jnp.int32),
        pltpu.VMEM((128, 16), jnp.float32),
        pltpu.VMEM((128, 16), jnp.float32),
        pltpu.VMEM((128, 16), jnp.float32),
        pltpu.VMEM((_HEADS, 128), jnp.float32),
        pltpu.VMEM((128, 16), jnp.float32),
        pltpu.VMEM((128, 16), jnp.float32),
        pltpu.VMEM((128, 16), jnp.float32),
        pltpu.VMEM((_HEADS, 128), jnp.float32),
    ],
)
def _sc_gat(hp_hbm, asrc_hbm, atrg_hbm, src_hbm, trg_hbm, m_hbm,
            out_hbm, exp_hbm, denom_sp, agg_sp, srcb, trgb, offb,
            sbuf, tbuf, ebuf, ehead, ewt, mbuf,
            offb2, semi, semg, semsc,
            srcb2, trgb2, ewt2, semi2, semg2, semsc2,
            offb3, offb4, sbuf2, tbuf2, ebuf2, ehead2,
            srcb3, trgb3, ewt3, srcb4, trgb4, ewt4,
            semi3, semg3, semsc3, semi4, semg4, semsc4,
            offb5, offb6, offb7, offb8, sbuf3, tbuf3, ebuf3, ehead3,
            sbuf4, tbuf4, ebuf4, ehead4):
    b = lax.axis_index("c")
    s = lax.axis_index("s")
    iota = lax.iota(jnp.int32, 16)
    r0 = iota >> 2
    c0 = iota & 3
    zv = jnp.zeros((16,), jnp.float32)

    def _zfill(i, carry):
        tbuf[i, :] = zv
        ebuf[i, :] = zv
        ebuf2[i, :] = zv
        ebuf3[i, :] = zv
        ebuf4[i, :] = zv
        return carry

    lax.fori_loop(0, 128, _zfill, 0)

    pltpu.sync_copy(m_hbm.at[pl.ds(b * 32, 32)], mbuf)
    msum = mbuf[pl.ds(0, 16)] + mbuf[pl.ds(16, 16)]
    mvec = jnp.maximum(msum, 0.2 * msum)

    def _zden(j, carry):
        g = j * 16 + s

        @pl.when(g < _NCH)
        def _():
            pltpu.sync_copy(tbuf, denom_sp.at[pl.ds(g * 128, 128)])
        return carry

    lax.fori_loop(0, 25, _zden, 0)
    plsc.subcore_barrier()

    noff = b * _N

    def _p1compute(sbuf_, tbuf_, ebuf_):
        for k in range(32):
            rows = r0 + 4 * k
            sv = plsc.load_gather(sbuf_, [rows, c0])
            tv = plsc.load_gather(tbuf_, [rows, c0])
            x = sv + tv
            ev = jnp.exp(jnp.maximum(x, 0.2 * x) - mvec)
            plsc.store_scatter(ebuf_, [rows, c0], ev)

    def _ehead_build(ebuf_, ehead_):
        for hd in range(_HEADS):
            hsel = jnp.full((16,), hd, jnp.int32)
            for k in range(8):
                ehead_[hd, pl.ds(k * 16, 16)] = plsc.load_gather(
                    ebuf_, [iota + 16 * k, hsel])

    p1s = [(srcb, trgb, offb, offb2, sbuf, tbuf, ebuf, ehead,
            semi, semg, semsc),
           (srcb2, trgb2, offb3, offb4, sbuf2, tbuf2, ebuf2, ehead2,
            semi2, semg2, semsc2),
           (srcb3, trgb3, offb5, offb6, sbuf3, tbuf3, ebuf3, ehead3,
            semi3, semg3, semsc3),
           (srcb4, trgb4, offb7, offb8, sbuf4, tbuf4, ebuf4, ehead4,
            semi4, semg4, semsc4)]

    def _p1(q, carry):
        gs = [(4 * q + t) * 16 + s for t in range(4)]
        for g_, (sb, tb, oS, oT, sB, tB, eB, eH, si, sg, sc) in zip(gs, p1s):
            @pl.when(g_ < _ECH)
            def _(g_=g_, sb=sb, tb=tb, si=si):
                base = b * _E + g_ * 128
                pltpu.async_copy(src_hbm.at[pl.ds(base, 128)], sb, si)
                pltpu.async_copy(trg_hbm.at[pl.ds(base, 128)], tb, si)
        for g_, (sb, tb, oS, oT, sB, tB, eB, eH, si, sg, sc) in zip(gs, p1s):
            @pl.when(g_ < _ECH)
            def _(g_=g_, sb=sb, tb=tb, oS=oS, oT=oT, sB=sB, tB=tB, si=si, sg=sg):
                base = b * _E + g_ * 128
                pltpu.make_async_copy(
                    src_hbm.at[pl.ds(base, 128)], sb, si).wait()
                pltpu.make_async_copy(
                    trg_hbm.at[pl.ds(base, 128)], tb, si).wait()
                for k in range(8):
                    oS[pl.ds(k * 16, 16)] = sb[pl.ds(k * 16, 16)] + noff
                    oT[pl.ds(k * 16, 16)] = tb[pl.ds(k * 16, 16)] + noff
                pltpu.async_copy(asrc_hbm.at[oS], sB, sg)
                pltpu.async_copy(atrg_hbm.at[oT], tB, sg)
        for g_, (sb, tb, oS, oT, sB, tB, eB, eH, si, sg, sc) in zip(gs, p1s):
            @pl.when(g_ < _ECH)
            def _(g_=g_, tb=tb, oS=oS, oT=oT, sB=sB, tB=tB, eB=eB, eH=eH,
                  sg=sg, sc=sc):
                pltpu.make_async_copy(asrc_hbm.at[oS], sB, sg).wait()
                pltpu.make_async_copy(atrg_hbm.at[oT], tB, sg).wait()
                _p1compute(sB, tB, eB)
                pltpu.async_copy(eB, denom_sp.at[tb], sc, add=True)
                _ehead_build(eB, eH)
                pltpu.async_copy(eH, exp_hbm.at[b * _ECH + g_], sg)
        for g_, (sb, tb, oS, oT, sB, tB, eB, eH, si, sg, sc) in zip(gs, p1s):
            @pl.when(g_ < _ECH)
            def _(g_=g_, tb=tb, eB=eB, eH=eH, sg=sg, sc=sc):
                pltpu.make_async_copy(eB, denom_sp.at[tb], sc).wait()
                pltpu.make_async_copy(eH, exp_hbm.at[b * _ECH + g_], sg).wait()
        return carry

    lax.fori_loop(0, 98, _p1, 0)
    plsc.subcore_barrier()

    def _head_pass(hd, hcarry):
        def _tz(i, carry):
            tbuf[i, :] = zv
            return carry

        lax.fori_loop(0, 128, _tz, 0)

        def _zagg(j, carry):
            g = j * 16 + s

            @pl.when(g < _NCH)
            def _():
                pltpu.sync_copy(tbuf, agg_sp.at[pl.ds(g * 128, 128)])
            return carry

        lax.fori_loop(0, 25, _zagg, 0)
        plsc.subcore_barrier()

        hoff = (b * _HEADS + hd) * _N
        eoff = (b * _HEADS + hd) * _E

        def _scale(ewt_, hpbuf_):
            def _wm(e_i, carry):
                wv = plsc.load_gather(ewt_, [jnp.full((16,), e_i, jnp.int32)])
                hpbuf_[e_i, :] = hpbuf_[e_i, :] * wv
                return carry

            lax.fori_loop(0, 128, _wm, 0, unroll=8)

        p2s = [(srcb, trgb, offb, ewt, sbuf, semi, semg, semsc),
               (srcb2, trgb2, offb2, ewt2, sbuf2, semi2, semg2, semsc2),
               (srcb3, trgb3, offb3, ewt3, sbuf3, semi3, semg3, semsc3),
               (srcb4, trgb4, offb4, ewt4, sbuf4, semi4, semg4, semsc4)]

        def _p2(q, carry):
            gs = [(4 * q + t) * 16 + s for t in range(4)]
            for g_, (sb, tb, ob, ew, hb, si, sg, sc) in zip(gs, p2s):
                @pl.when(g_ < _ECH)
                def _(g_=g_, sb=sb, tb=tb, ew=ew, si=si):
                    base = b * _E + g_ * 128
                    pltpu.async_copy(src_hbm.at[pl.ds(base, 128)], sb, si)
                    pltpu.async_copy(trg_hbm.at[pl.ds(base, 128)], tb, si)
                    pltpu.async_copy(exp_hbm.at[b * _ECH + g_, hd], ew, si)
            for g_, (sb, tb, ob, ew, hb, si, sg, sc) in zip(gs, p2s):
                @pl.when(g_ < _ECH)
                def _(g_=g_, sb=sb, tb=tb, ob=ob, ew=ew, hb=hb, si=si, sg=sg):
                    base = b * _E + g_ * 128
                    pltpu.make_async_copy(
                        src_hbm.at[pl.ds(base, 128)], sb, si).wait()
                    pltpu.make_async_copy(
                        trg_hbm.at[pl.ds(base, 128)], tb, si).wait()
                    pltpu.make_async_copy(
                        exp_hbm.at[b * _ECH + g_, hd], ew, si).wait()
                    for k in range(8):
                        ob[pl.ds(k * 16, 16)] = sb[pl.ds(k * 16, 16)] + hoff
                    pltpu.async_copy(hp_hbm.at[ob], hb, sg)
            for g_, (sb, tb, ob, ew, hb, si, sg, sc) in zip(gs, p2s):
                @pl.when(g_ < _ECH)
                def _(g_=g_, tb=tb, ob=ob, ew=ew, hb=hb, sg=sg, sc=sc):
                    pltpu.make_async_copy(hp_hbm.at[ob], hb, sg).wait()
                    _scale(ew, hb)
                    pltpu.async_copy(hb, agg_sp.at[tb], sc, add=True)
            for g_, (sb, tb, ob, ew, hb, si, sg, sc) in zip(gs, p2s):
                @pl.when(g_ < _ECH)
                def _(g_=g_, tb=tb, hb=hb, sc=sc):
                    pltpu.make_async_copy(hb, agg_sp.at[tb], sc).wait()
            return carry

        lax.fori_loop(0, 98, _p2, 0)
        plsc.subcore_barrier()

        def _fin(j, carry):
            g = j * 16 + s

            @pl.when(g < _NFULL)
            def _():
                nb = g * 128
                pltpu.sync_copy(agg_sp.at[pl.ds(nb, 128)], tbuf)
                pltpu.sync_copy(denom_sp.at[pl.ds(nb, 128)], tbuf2)
                hsel = jnp.full((16,), hd, jnp.int32)

                def _ndiv(i, carry):
                    dv = plsc.load_gather(tbuf2, [jnp.full((16,), i, jnp.int32), hsel])
                    tbuf[i, :] = tbuf[i, :] / (dv + 1e-16)
                    return carry

                lax.fori_loop(0, 128, _ndiv, 0, unroll=8)
                pltpu.sync_copy(
                    tbuf, out_hbm.at[pl.ds((b * _HEADS + hd) * _N + nb, 128)])
            return carry

        lax.fori_loop(0, 25, _fin, 0)

        @pl.when(s == _TAIL_TILE)
        def _():
            nb = _NFULL * 128
            pltpu.sync_copy(agg_sp.at[pl.ds(nb, _TAIL)], tbuf.at[pl.ds(0, _TAIL)])
            pltpu.sync_copy(denom_sp.at[pl.ds(nb, _TAIL)], tbuf2.at[pl.ds(0, _TAIL)])
            hsel = jnp.full((16,), hd, jnp.int32)

            def _ndiv2(i, carry):
                dv = plsc.load_gather(tbuf2, [jnp.full((16,), i, jnp.int32), hsel])
                tbuf[i, :] = tbuf[i, :] / (dv + 1e-16)
                return carry

            lax.fori_loop(0, _TAIL, _ndiv2, 0, unroll=8)
            pltpu.sync_copy(
                tbuf.at[pl.ds(0, _TAIL)],
                out_hbm.at[pl.ds((b * _HEADS + hd) * _N + nb, _TAIL)])

        plsc.subcore_barrier()
        return hcarry

    lax.fori_loop(0, _HEADS, _head_pass, 0)


def kernel(h, adjs, W, attn_src, attn_trg):
    src = adjs[:, 0, :].astype(jnp.int32).reshape(_B * _E)
    trg = adjs[:, 1, :].astype(jnp.int32).reshape(_B * _E)
    hp4, asrc, atrg, mpair = _proj(h, W, attn_src, attn_trg)
    out_flat, _ = _sc_gat(
        hp4.reshape(_B * _HEADS * _N, _OUT_F),
        asrc.reshape(_B * _N, 16),
        atrg.reshape(_B * _N, 16),
        src, trg,
        mpair[:, :, 0, :16].reshape(_B * 32),
    )
    return out_flat.reshape(_B, _HEADS, _N, _OUT_F)
